# baseline (speedup 1.0000x reference)
import functools

import jax
import jax.numpy as jnp
from jax import lax
from jax.experimental import pallas as pl
from jax.experimental.pallas import tpu as pltpu, tpu_sc as plsc

N = 10000
E = 160000
EMB = 64
H = 128
VOCAB = 257

NC, NS = 2, 16
NW = NC * NS
NPAD = 10112
SLICE = NPAD // NS
EPAD = 163840
EPT = EPAD // NW
CH = 128
NCHUNK = EPT // CH
NXC = NPAD // CH
DEPTH = 3

_mesh = functools.partial(plsc.VectorSubcoreMesh,
                          core_axis_name="c", subcore_axis_name="s")


def _sc_embed_counts(feats, embs, edst_offs, zcol, ones):
    @functools.partial(
        pl.kernel,
        out_type=(jax.ShapeDtypeStruct((3, NPAD, H), jnp.float32),
                  jax.ShapeDtypeStruct((NC * 3 * NPAD,), jnp.float32)),
        mesh=_mesh(),
        scratch_types=[
            pltpu.VMEM((CH,), jnp.int32),
            pltpu.VMEM((CH, H), jnp.float32),
            pltpu.VMEM((CH,), jnp.int32),
            pltpu.VMEM((CH,), jnp.float32),
            pltpu.VMEM_SHARED((3 * NPAD,), jnp.float32),
            pltpu.SemaphoreType.DMA,
        ],
    )
    def k(f0, f1, f2, e0, e1, e2, d0, d1, d2, zc, on,
          x_out, cnt_out, fidx, xrows, didx, ones_v, acc, sem):
        cid = lax.axis_index("c")
        sid = lax.axis_index("s")
        wid = cid * NS + sid
        feat_refs = (f0, f1, f2)
        emb_refs = (e0, e1, e2)
        dst_refs = (d0, d1, d2)
        sync_copy = pltpu.sync_copy

        sync_copy(on, ones_v)

        def emb_chunk(r, c):
            sync_copy(feat_refs[r].at[c], fidx)
            pltpu.async_copy(emb_refs[r].at[fidx], xrows, sem).wait()
            sync_copy(xrows, x_out.at[r, pl.ds(c * CH, CH)])
            sync_copy(zc, acc.at[pl.ds(r * NPAD + c * CH, CH)])

        for r in range(3):
            emb_chunk(r, 2 * wid)
            emb_chunk(r, 2 * wid + 1)

            @pl.when(wid < NXC - 2 * NW)
            def _(r=r):
                emb_chunk(r, 2 * NW + wid)
        plsc.subcore_barrier()
        for r in range(3):
            def body(j, carry, r=r):
                sync_copy(dst_refs[r].at[wid, j], didx)
                sync_copy(ones_v, acc.at[didx], add=True)
                return carry
            lax.fori_loop(0, NCHUNK, body, 0)
        plsc.subcore_barrier()

        @pl.when(sid == 0)
        def _():
            sync_copy(acc, cnt_out.at[pl.ds(cid * 3 * NPAD, 3 * NPAD)])

    return k(feats[0], feats[1], feats[2], embs[0], embs[1], embs[2],
             edst_offs[0], edst_offs[1], edst_offs[2], zcol, ones)


def _sc_aggregate(W):
    @functools.partial(
        pl.kernel,
        out_type=jax.ShapeDtypeStruct((NC, 3, NPAD, W), jnp.float32),
        mesh=_mesh(),
        scratch_types=[
            [pltpu.VMEM((CH,), jnp.int32)] * DEPTH,
            [pltpu.VMEM((CH,), jnp.int32)] * DEPTH,
            [pltpu.VMEM((CH, W), jnp.float32)] * DEPTH,
            pltpu.VMEM_SHARED((NPAD, W), jnp.float32),
            [pltpu.SemaphoreType.DMA] * DEPTH,
            [pltpu.SemaphoreType.DMA] * DEPTH,
        ],
    )
    def k(h_all, s0, s1, s2, d0, d1, d2, zrows,
          out, sring, dring, rows, acc, gsem, isem):
        cid = lax.axis_index("c")
        sid = lax.axis_index("s")
        wid = cid * NS + sid
        base = sid * SLICE
        src_refs = (s0, s1, s2)
        dst_refs = (d0, d1, d2)
        sync_copy = pltpu.sync_copy

        for r in range(3):
            sync_copy(zrows, acc.at[pl.ds(base, SLICE)])
            plsc.subcore_barrier()

            hr = h_all.at[r]
            sref = src_refs[r]
            dref = dst_refs[r]

            def load_idx(j, k, sref=sref, dref=dref):
                pltpu.async_copy(sref.at[wid, j], sring[k], isem[k])
                pltpu.async_copy(dref.at[wid, j], dring[k], isem[k])

            def wait_idx(j, k, sref=sref, dref=dref):
                pltpu.make_async_copy(sref.at[wid, j], sring[k],
                                      isem[k]).wait()
                pltpu.make_async_copy(dref.at[wid, j], dring[k],
                                      isem[k]).wait()

            def fire_g(k, hr=hr):
                pltpu.async_copy(hr.at[sring[k]], rows[k], gsem[k])

            def wait_g(k, hr=hr):
                pltpu.make_async_copy(hr.at[sring[k]], rows[k],
                                      gsem[k]).wait()

            def fire_s(k):
                sync_copy(rows[k], acc.at[dring[k]], add=True)

            for k in range(DEPTH):
                load_idx(k, k)
            for k in range(2):
                wait_idx(k, k)
                fire_g(k)

            def group(gidx, carry):
                j0 = DEPTH * gidx
                for u in range(DEPTH):
                    j = j0 + u

                    @pl.when(j + 2 < NCHUNK)
                    def _(j=j, k2=(u + 2) % DEPTH):
                        wait_idx(j + 2, k2)
                        fire_g(k2)
                    wait_g(u)
                    fire_s(u)

                    @pl.when(j + DEPTH < NCHUNK)
                    def _(j=j, u=u):
                        load_idx(j + DEPTH, u)
                return carry
            lax.fori_loop(0, NCHUNK // DEPTH, group, 0)
            wait_g(0)
            fire_s(0)
            plsc.subcore_barrier()
            sync_copy(acc.at[pl.ds(base, SLICE)],
                      out.at[cid, r, pl.ds(base, SLICE)])
            plsc.subcore_barrier()

    return k


def _tc_layer(Win, BN=632):
    NB = NPAD // BN

    def body(h_ref, p_ref, cnt_ref, ws_ref, wn_ref, b_ref, a_ref,
             g_ref, be_ref, y_ref, ms_ref):
        b = pl.program_id(1)
        h = h_ref[0]
        pp = p_ref[0, 0] + p_ref[1, 0]
        inv = 1.0 / jnp.maximum(cnt_ref[0], 1.0)
        agg = pp * inv
        rst = (jnp.dot(h, ws_ref[0], preferred_element_type=jnp.float32)
               + jnp.dot(agg, wn_ref[0], preferred_element_type=jnp.float32)
               + b_ref[0])
        rst = jnp.where(rst > 0, rst, a_ref[0] * rst)
        y = g_ref[0] * rst + be_ref[0]
        y_ref[0] = y
        rows = b * BN + lax.broadcasted_iota(jnp.int32, (BN, H), 0)
        s = jnp.sum(jnp.where(rows < N, y, 0.0), axis=0, keepdims=True)

        @pl.when(b == 0)
        def _():
            ms_ref[0] = s

        @pl.when(b != 0)
        def _():
            ms_ref[0] += s

    return pl.pallas_call(
        body,
        grid=(3, NB),
        in_specs=[
            pl.BlockSpec((1, BN, Win), lambda r, b: (r, b, 0)),
            pl.BlockSpec((NC, 1, BN, Win), lambda r, b: (0, r, b, 0)),
            pl.BlockSpec((1, BN, 1), lambda r, b: (r, b, 0)),
            pl.BlockSpec((1, Win, H), lambda r, b: (r, 0, 0)),
            pl.BlockSpec((1, Win, H), lambda r, b: (r, 0, 0)),
            pl.BlockSpec((1, 1, H), lambda r, b: (r, 0, 0)),
            pl.BlockSpec((1, 1, H), lambda r, b: (r, 0, 0)),
            pl.BlockSpec((1, 1, H), lambda r, b: (r, 0, 0)),
            pl.BlockSpec((1, 1, H), lambda r, b: (r, 0, 0)),
        ],
        out_specs=[
            pl.BlockSpec((1, BN, H), lambda r, b: (r, b, 0)),
            pl.BlockSpec((1, 1, H), lambda r, b: (r, 0, 0)),
        ],
        out_shape=[jax.ShapeDtypeStruct((3, NPAD, H), jnp.float32),
                   jax.ShapeDtypeStruct((3, 1, H), jnp.float32)],
    )


def kernel(feat_h, feat_p, feat_hp, eidx_h, eidx_p, eidx_hp,
           emb_h, emb_p, emb_hp, Ws1, Wn1, b1, a1, g1, be1,
           Ws, Wn, b, a, g, be):
    i32 = jnp.int32
    npad_feat = jnp.zeros((NPAD - N,), i32)
    feats = [jnp.concatenate([f.astype(i32), npad_feat]).reshape(NXC, CH)
             for f in (feat_h, feat_p, feat_hp)]
    pad_ar = jnp.arange(EPAD - E, dtype=i32)
    pad_src = pad_ar % 64
    pad_dst = N + pad_ar % (NPAD - N)
    esrcs, edsts, edst_offs = [], [], []
    for r, eidx in enumerate((eidx_h, eidx_p, eidx_hp)):
        esrcs.append(jnp.concatenate([eidx[0].astype(i32), pad_src])
                     .reshape(NW, NCHUNK, CH))
        dst = jnp.concatenate([eidx[1].astype(i32), pad_dst])
        edsts.append(dst.reshape(NW, NCHUNK, CH))
        edst_offs.append((dst + r * NPAD).reshape(NW, NCHUNK, CH))

    zcol = jnp.zeros((CH,), jnp.float32)
    ones = jnp.ones((CH,), jnp.float32)

    zpadE = jnp.zeros((VOCAB, H - EMB), jnp.float32)
    embs = tuple(jnp.concatenate([t, zpadE], axis=1)
                 for t in (emb_h, emb_p, emb_hp))
    zpadW = jnp.zeros((3, H - EMB, H), jnp.float32)
    Ws1p = jnp.concatenate([Ws1, zpadW], axis=1)
    Wn1p = jnp.concatenate([Wn1, zpadW], axis=1)

    x, cnt = _sc_embed_counts(feats, embs, edst_offs, zcol, ones)
    cnt2 = cnt.reshape(NC, 3, NPAD)
    cnt_sum = (cnt2[0] + cnt2[1]).reshape(3, NPAD, 1)

    agg128 = _sc_aggregate(H)
    z128 = jnp.zeros((SLICE, H), jnp.float32)
    tc_layer = _tc_layer(H)

    def sage(h, Wsl, Wnl, bl, al, gl, bel):
        p = agg128(h,
                   esrcs[0], esrcs[1], esrcs[2],
                   edsts[0], edsts[1], edsts[2], z128)
        y, ms = tc_layer(
            h, p, cnt_sum,
            Wsl, Wnl,
            bl.reshape(3, 1, H), al.reshape(3, 1, H),
            gl.reshape(3, 1, H), bel.reshape(3, 1, H))
        return y, ms

    means = []
    hcur, msum = sage(x, Ws1p, Wn1p, b1, a1, g1, be1)
    means.append(msum)
    for l in range(3):
        hcur, msum = sage(hcur, Ws[l], Wn[l], b[l], a[l], g[l], be[l])
        means.append(msum)

    m = jnp.stack(means)[:, :, 0, :] / N
    g_vec = jnp.transpose(m, (1, 0, 2)).reshape(1, 3 * 4 * H)
    return g_vec


# speedup vs baseline: 1.1087x; 1.1087x over previous
import functools

import jax
import jax.numpy as jnp
from jax import lax
from jax.experimental import pallas as pl
from jax.experimental.pallas import tpu as pltpu, tpu_sc as plsc

N = 10000
E = 160000
EMB = 64
H = 128
VOCAB = 257
VPAD = 384

NC, NS = 2, 16
NW = NC * NS
NPAD = 10240
SLICE = NPAD // NS
EPAD = 163840
EPT = EPAD // NW
CH = 128
NCHUNK = EPT // CH
NXC = NPAD // CH

_mesh = functools.partial(plsc.VectorSubcoreMesh,
                          core_axis_name="c", subcore_axis_name="s")


def _sc_embed_counts(feats, embs, edst_offs, zcol, ones):
    @functools.partial(
        pl.kernel,
        out_type=(jax.ShapeDtypeStruct((3, NPAD, H), jnp.float32),
                  jax.ShapeDtypeStruct((NC * 3 * NPAD,), jnp.float32)),
        mesh=_mesh(),
        scratch_types=[
            pltpu.VMEM((CH,), jnp.int32),
            pltpu.VMEM((CH, H), jnp.float32),
            pltpu.VMEM((3, NCHUNK, CH), jnp.int32),
            pltpu.VMEM((CH,), jnp.float32),
            pltpu.VMEM_SHARED((3 * NPAD,), jnp.float32),
            pltpu.VMEM_SHARED((3, VPAD, H), jnp.float32),
            pltpu.SemaphoreType.DMA,
            pltpu.SemaphoreType.DMA,
        ],
    )
    def k(f0, f1, f2, e0, e1, e2, d0, d1, d2, zc, on,
          x_out, cnt_out, fidx, xrows, didxb, ones_v, acc, emsp,
          gsem, ssem):
        cid = lax.axis_index("c")
        sid = lax.axis_index("s")
        wid = cid * NS + sid
        feat_refs = (f0, f1, f2)
        emb_refs = (e0, e1, e2)
        dst_refs = (d0, d1, d2)
        sync_copy = pltpu.sync_copy
        RPT = VPAD // NS

        sync_copy(on, ones_v)
        for r in range(3):
            sync_copy(emb_refs[r].at[pl.ds(sid * RPT, RPT)],
                      emsp.at[r, pl.ds(sid * RPT, RPT)])
            sync_copy(dst_refs[r].at[wid], didxb.at[r])
            sync_copy(zc, acc.at[pl.ds(r * NPAD + sid * SLICE, SLICE)])
        plsc.subcore_barrier()

        def emb_chunk(r, c):
            sync_copy(feat_refs[r].at[c], fidx)
            pltpu.async_copy(emsp.at[r].at[fidx], xrows, gsem).wait()
            sync_copy(xrows, x_out.at[r, pl.ds(c * CH, CH)])

        for r in range(3):
            for j in range(NCHUNK):
                pltpu.async_copy(ones_v, acc.at[didxb.at[r, j]], ssem,
                                 add=True)
            emb_chunk(r, 2 * wid)
            emb_chunk(r, 2 * wid + 1)

            @pl.when(wid < NXC - 2 * NW)
            def _(r=r):
                emb_chunk(r, 2 * NW + wid)
        for r in range(3):
            for j in range(NCHUNK):
                pltpu.make_async_copy(ones_v, acc.at[didxb.at[r, j]],
                                      ssem).wait()
        plsc.subcore_barrier()

        @pl.when(sid == 0)
        def _():
            sync_copy(acc, cnt_out.at[pl.ds(cid * 3 * NPAD, 3 * NPAD)])

    return k(feats[0], feats[1], feats[2], embs[0], embs[1], embs[2],
             edst_offs[0], edst_offs[1], edst_offs[2], zcol, ones)


def _sc_aggregate(W):
    @functools.partial(
        pl.kernel,
        out_type=jax.ShapeDtypeStruct((NC, 3, NPAD, W), jnp.float32),
        mesh=_mesh(),
        scratch_types=[
            pltpu.VMEM((NCHUNK, CH), jnp.int32),
            pltpu.VMEM((NCHUNK, CH), jnp.int32),
            pltpu.VMEM((CH, W), jnp.float32),
            pltpu.VMEM((CH, W), jnp.float32),
            pltpu.VMEM_SHARED((NPAD, W), jnp.float32),
            pltpu.SemaphoreType.DMA,
            pltpu.SemaphoreType.DMA,
        ],
    )
    def k(h_all, s0, s1, s2, d0, d1, d2, zrows,
          out, sidx, didx, rows0, rows1, acc, sem0, sem1):
        cid = lax.axis_index("c")
        sid = lax.axis_index("s")
        wid = cid * NS + sid
        base = sid * SLICE
        src_refs = (s0, s1, s2)
        dst_refs = (d0, d1, d2)
        sync_copy = pltpu.sync_copy
        rows = (rows0, rows1)
        sems = (sem0, sem1)

        for r in range(3):
            sync_copy(zrows, acc.at[pl.ds(base, SLICE)])
            sync_copy(src_refs[r].at[wid], sidx)
            sync_copy(dst_refs[r].at[wid], didx)
            plsc.subcore_barrier()

            hr = h_all.at[r]

            def fire_g(j, b, hr=hr):
                pltpu.async_copy(hr.at[sidx.at[j]], rows[b], sems[b])

            def wait_g(j, b, hr=hr):
                pltpu.make_async_copy(hr.at[sidx.at[j]], rows[b],
                                      sems[b]).wait()

            fire_g(0, 0)

            def pair(jj, carry):
                j0 = 2 * jj
                for u in range(2):
                    j = j0 + u

                    @pl.when(j + 1 < NCHUNK)
                    def _(j=j, u=u):
                        fire_g(j + 1, (u + 1) % 2)
                    wait_g(j, u)
                    sync_copy(rows[u], acc.at[didx.at[j]], add=True)
                return carry
            lax.fori_loop(0, NCHUNK // 2, pair, 0)
            plsc.subcore_barrier()
            sync_copy(acc.at[pl.ds(base, SLICE)],
                      out.at[cid, r, pl.ds(base, SLICE)])
            plsc.subcore_barrier()

    return k


def _tc_layer(Win, BN=512):
    NB = NPAD // BN

    def body(h_ref, p_ref, cnt_ref, ws_ref, wn_ref, b_ref, a_ref,
             g_ref, be_ref, y_ref, ms_ref):
        b = pl.program_id(1)
        h = h_ref[0]
        pp = p_ref[0, 0] + p_ref[1, 0]
        inv = 1.0 / jnp.maximum(cnt_ref[0], 1.0)
        agg = pp * inv
        rst = (jnp.dot(h, ws_ref[0], preferred_element_type=jnp.float32)
               + jnp.dot(agg, wn_ref[0], preferred_element_type=jnp.float32)
               + b_ref[0])
        rst = jnp.where(rst > 0, rst, a_ref[0] * rst)
        y = g_ref[0] * rst + be_ref[0]
        y_ref[0] = y
        rows = b * BN + lax.broadcasted_iota(jnp.int32, (BN, H), 0)
        s = jnp.sum(jnp.where(rows < N, y, 0.0), axis=0, keepdims=True)

        @pl.when(b == 0)
        def _():
            ms_ref[0] = s

        @pl.when(b != 0)
        def _():
            ms_ref[0] += s

    return pl.pallas_call(
        body,
        grid=(3, NB),
        in_specs=[
            pl.BlockSpec((1, BN, Win), lambda r, b: (r, b, 0)),
            pl.BlockSpec((NC, 1, BN, Win), lambda r, b: (0, r, b, 0)),
            pl.BlockSpec((1, BN, 1), lambda r, b: (r, b, 0)),
            pl.BlockSpec((1, Win, H), lambda r, b: (r, 0, 0)),
            pl.BlockSpec((1, Win, H), lambda r, b: (r, 0, 0)),
            pl.BlockSpec((1, 1, H), lambda r, b: (r, 0, 0)),
            pl.BlockSpec((1, 1, H), lambda r, b: (r, 0, 0)),
            pl.BlockSpec((1, 1, H), lambda r, b: (r, 0, 0)),
            pl.BlockSpec((1, 1, H), lambda r, b: (r, 0, 0)),
        ],
        out_specs=[
            pl.BlockSpec((1, BN, H), lambda r, b: (r, b, 0)),
            pl.BlockSpec((1, 1, H), lambda r, b: (r, 0, 0)),
        ],
        out_shape=[jax.ShapeDtypeStruct((3, NPAD, H), jnp.float32),
                   jax.ShapeDtypeStruct((3, 1, H), jnp.float32)],
    )


def kernel(feat_h, feat_p, feat_hp, eidx_h, eidx_p, eidx_hp,
           emb_h, emb_p, emb_hp, Ws1, Wn1, b1, a1, g1, be1,
           Ws, Wn, b, a, g, be):
    i32 = jnp.int32
    npad_feat = jnp.zeros((NPAD - N,), i32)
    feats = [jnp.concatenate([f.astype(i32), npad_feat]).reshape(NXC, CH)
             for f in (feat_h, feat_p, feat_hp)]
    pad_ar = jnp.arange(EPAD - E, dtype=i32)
    pad_src = pad_ar % 64
    pad_dst = N + pad_ar % (NPAD - N)
    esrcs, edsts, edst_offs = [], [], []
    for r, eidx in enumerate((eidx_h, eidx_p, eidx_hp)):
        esrcs.append(jnp.concatenate([eidx[0].astype(i32), pad_src])
                     .reshape(NW, NCHUNK, CH))
        dst = jnp.concatenate([eidx[1].astype(i32), pad_dst])
        edsts.append(dst.reshape(NW, NCHUNK, CH))
        edst_offs.append((dst + r * NPAD).reshape(NW, NCHUNK, CH))

    zcol = jnp.zeros((SLICE,), jnp.float32)
    ones = jnp.ones((CH,), jnp.float32)

    embs = []
    for t in (emb_h, emb_p, emb_hp):
        tp = jnp.zeros((VPAD, H), jnp.float32)
        embs.append(tp.at[:VOCAB, :EMB].set(t))
    zpadW = jnp.zeros((3, H - EMB, H), jnp.float32)
    Ws1p = jnp.concatenate([Ws1, zpadW], axis=1)
    Wn1p = jnp.concatenate([Wn1, zpadW], axis=1)

    x, cnt = _sc_embed_counts(feats, embs, edst_offs, zcol, ones)
    cnt2 = cnt.reshape(NC, 3, NPAD)
    cnt_sum = (cnt2[0] + cnt2[1]).reshape(3, NPAD, 1)

    agg128 = _sc_aggregate(H)
    z128 = jnp.zeros((SLICE, H), jnp.float32)
    tc_layer = _tc_layer(H)

    def sage(h, Wsl, Wnl, bl, al, gl, bel):
        p = agg128(h,
                   esrcs[0], esrcs[1], esrcs[2],
                   edsts[0], edsts[1], edsts[2], z128)
        y, ms = tc_layer(
            h, p, cnt_sum,
            Wsl, Wnl,
            bl.reshape(3, 1, H), al.reshape(3, 1, H),
            gl.reshape(3, 1, H), bel.reshape(3, 1, H))
        return y, ms

    means = []
    hcur, msum = sage(x, Ws1p, Wn1p, b1, a1, g1, be1)
    means.append(msum)
    for l in range(3):
        hcur, msum = sage(hcur, Ws[l], Wn[l], b[l], a[l], g[l], be[l])
        means.append(msum)

    m = jnp.stack(means)[:, :, 0, :] / N
    g_vec = jnp.transpose(m, (1, 0, 2)).reshape(1, 3 * 4 * H)
    return g_vec
